# baseline (speedup 1.0000x reference)
import functools

import jax
import jax.numpy as jnp
from jax.experimental import pallas as pl
from jax.experimental.pallas import tpu as pltpu

_NUM_BEES = 40
_BLK = 1024
_PAIR = 2 * _BLK


def _bee_kernel(p_ref, psq_ref, bestpos_ref, bestfit_ref, xa_ref, xb_ref,
                out_ref, xs_ref, acc_ref, add_ref, *, inv_n):
    phase = pl.program_id(0)
    i = pl.program_id(1)

    @pl.when(phase == 0)
    def _():
        @pl.when(i == 0)
        def _():
            acc_ref[...] = jnp.zeros_like(acc_ref)

        p_hi = p_ref[...].astype(jnp.bfloat16)
        for half, x_ref in ((0, xa_ref), (1, xb_ref)):
            x = x_ref[...]
            xs_ref[pl.ds(i * _PAIR + half * _BLK, _BLK), :] = x
            x_hi = x.astype(jnp.bfloat16)
            dot = jax.lax.dot_general(
                x_hi, p_hi, dimension_numbers=(((1,), (1,)), ((), ())),
                preferred_element_type=jnp.float32)
            x_sq = jnp.sum(x * x, axis=1, keepdims=True)
            sq = jnp.maximum(x_sq - 2.0 * dot + psq_ref[...], 0.0)
            acc_ref[...] += jnp.sum(jnp.sqrt(sq), axis=0, keepdims=True)

    @pl.when((phase == 1) & (i == 0))
    def _():
        sums = acc_ref[...]
        max_sum = jnp.max(sums)
        iota = jax.lax.broadcasted_iota(jnp.int32, (1, _NUM_BEES), 1)
        idx = jnp.min(jnp.where(sums == max_sum, iota, _NUM_BEES))
        onehot = jax.lax.broadcasted_iota(jnp.int32, (_NUM_BEES, 1), 0) == idx
        chosen = jnp.sum(jnp.where(onehot, p_ref[...], 0.0), axis=0,
                         keepdims=True)
        better = max_sum * inv_n > bestfit_ref[0, 0]
        add_ref[...] = jnp.where(better, chosen, bestpos_ref[...])

    @pl.when(phase == 1)
    def _():
        out_ref[...] = xs_ref[pl.ds(i * _PAIR, _PAIR), :] + add_ref[...]


def kernel(x, bee_positions, bee_fitness, best_position, best_fitness,
           exploration_radius, exploitation_radius):
    B, S, H = x.shape
    n_rows = B * S
    xr = x.reshape(n_rows, H)
    n_pair = n_rows // _PAIR
    psq = jnp.sum(bee_positions * bee_positions, axis=1, keepdims=True).T

    body = functools.partial(_bee_kernel, inv_n=1.0 / n_rows)

    out = pl.pallas_call(
        body,
        grid=(2, n_pair),
        in_specs=[
            pl.BlockSpec((_NUM_BEES, H), lambda p, i: (0, 0)),
            pl.BlockSpec((1, _NUM_BEES), lambda p, i: (0, 0)),
            pl.BlockSpec((1, H), lambda p, i: (0, 0)),
            pl.BlockSpec((1, 1), lambda p, i: (0, 0)),
            pl.BlockSpec((_BLK, H),
                         lambda p, i: (2 * i * (1 - p) + (2 * n_pair - 2) * p,
                                       0)),
            pl.BlockSpec((_BLK, H),
                         lambda p, i: ((2 * i + 1) * (1 - p)
                                       + (2 * n_pair - 1) * p, 0)),
        ],
        out_specs=pl.BlockSpec((_PAIR, H), lambda p, i: (i * p, 0)),
        out_shape=jax.ShapeDtypeStruct((n_rows, H), jnp.float32),
        scratch_shapes=[
            pltpu.VMEM((n_rows, H), jnp.float32),
            pltpu.VMEM((1, _NUM_BEES), jnp.float32),
            pltpu.VMEM((1, H), jnp.float32),
        ],
    )(bee_positions, psq, best_position.reshape(1, H),
      best_fitness.reshape(1, 1), xr, xr)

    return out.reshape(B, S, H)


# speedup vs baseline: 1.0995x; 1.0995x over previous
import functools

import jax
import jax.numpy as jnp
from jax.experimental import pallas as pl
from jax.experimental.pallas import tpu as pltpu

_NUM_BEES = 40
_BLK = 1024


def _bee_kernel(p_ref, psq_ref, bestpos_ref, bestfit_ref, x_ref, out_ref,
                xs_ref, acc_ref, add_ref, *, inv_n):
    phase = pl.program_id(0)
    i = pl.program_id(1)

    @pl.when(phase == 0)
    def _():
        x = x_ref[...]
        xs_ref[pl.ds(i * _BLK, _BLK), :] = x
        x_hi = x.astype(jnp.bfloat16)
        p_hi = p_ref[...].astype(jnp.bfloat16)
        dot = jax.lax.dot_general(
            x_hi, p_hi, dimension_numbers=(((1,), (1,)), ((), ())),
            preferred_element_type=jnp.float32)
        x_sq = jnp.sum(x * x, axis=1, keepdims=True)
        sq = jnp.maximum(x_sq - 2.0 * dot + psq_ref[...], 0.0)
        partial = jnp.sum(jnp.sqrt(sq), axis=0, keepdims=True)

        @pl.when(i == 0)
        def _():
            acc_ref[...] = jnp.zeros_like(acc_ref)

        acc_ref[...] += partial

    @pl.when((phase == 1) & (i == 0))
    def _():
        sums = acc_ref[...]
        max_sum = jnp.max(sums)
        iota = jax.lax.broadcasted_iota(jnp.int32, (1, _NUM_BEES), 1)
        idx = jnp.min(jnp.where(sums == max_sum, iota, _NUM_BEES))
        onehot = jax.lax.broadcasted_iota(jnp.int32, (_NUM_BEES, 1), 0) == idx
        chosen = jnp.sum(jnp.where(onehot, p_ref[...], 0.0), axis=0,
                         keepdims=True)
        better = max_sum * inv_n > bestfit_ref[0, 0]
        add_ref[...] = jnp.where(better, chosen, bestpos_ref[...])

    @pl.when(phase == 1)
    def _():
        out_ref[...] = xs_ref[pl.ds(i * _BLK, _BLK), :] + add_ref[...]


def kernel(x, bee_positions, bee_fitness, best_position, best_fitness,
           exploration_radius, exploitation_radius):
    B, S, H = x.shape
    n_rows = B * S
    xr = x.reshape(n_rows, H)
    n_blk = n_rows // _BLK
    psq = jnp.sum(bee_positions * bee_positions, axis=1, keepdims=True).T

    body = functools.partial(_bee_kernel, inv_n=1.0 / n_rows)

    out = pl.pallas_call(
        body,
        grid=(2, n_blk),
        in_specs=[
            pl.BlockSpec((_NUM_BEES, H), lambda p, i: (0, 0)),
            pl.BlockSpec((1, _NUM_BEES), lambda p, i: (0, 0)),
            pl.BlockSpec((1, H), lambda p, i: (0, 0)),
            pl.BlockSpec((1, 1), lambda p, i: (0, 0)),
            pl.BlockSpec((_BLK, H),
                         lambda p, i: (i * (1 - p) + (n_blk - 1) * p, 0)),
        ],
        out_specs=pl.BlockSpec((_BLK, H), lambda p, i: (i * p, 0)),
        out_shape=jax.ShapeDtypeStruct((n_rows, H), jnp.float32),
        scratch_shapes=[
            pltpu.VMEM((n_rows, H), jnp.float32),
            pltpu.VMEM((1, _NUM_BEES), jnp.float32),
            pltpu.VMEM((1, H), jnp.float32),
        ],
    )(bee_positions, psq, best_position.reshape(1, H),
      best_fitness.reshape(1, 1), xr)

    return out.reshape(B, S, H)


# speedup vs baseline: 1.2110x; 1.1014x over previous
import functools

import jax
import jax.numpy as jnp
from jax.experimental import pallas as pl
from jax.experimental.pallas import tpu as pltpu

_NUM_BEES = 40
_BLK = 1024


def _bee_kernel(p_ref, bestpos_ref, bestfit_ref, x_ref, out_ref,
                xs_ref, acc_ref, add_ref, *, inv_n):
    phase = pl.program_id(0)
    i = pl.program_id(1)

    @pl.when(phase == 0)
    def _():
        x = x_ref[...]
        xs_ref[pl.ds(i * _BLK, _BLK), :] = x
        x_hi = x.astype(jnp.bfloat16)
        p_hi = p_ref[...].astype(jnp.bfloat16)
        dot = jax.lax.dot_general(
            x_hi, p_hi, dimension_numbers=(((1,), (1,)), ((), ())),
            preferred_element_type=jnp.float32)
        x_sq = jnp.sum(x * x, axis=1, keepdims=True)
        p = p_ref[...]
        psq = jnp.sum(p * p, axis=1)[None, :]
        sq = jnp.maximum(x_sq - 2.0 * dot + psq, 0.0)
        partial = jnp.sum(jnp.sqrt(sq), axis=0, keepdims=True)

        @pl.when(i == 0)
        def _():
            acc_ref[...] = jnp.zeros_like(acc_ref)

        acc_ref[...] += partial

    @pl.when((phase == 1) & (i == 0))
    def _():
        sums = acc_ref[...]
        max_sum = jnp.max(sums)
        iota = jax.lax.broadcasted_iota(jnp.int32, (1, _NUM_BEES), 1)
        idx = jnp.min(jnp.where(sums == max_sum, iota, _NUM_BEES))
        onehot = jax.lax.broadcasted_iota(jnp.int32, (_NUM_BEES, 1), 0) == idx
        chosen = jnp.sum(jnp.where(onehot, p_ref[...], 0.0), axis=0,
                         keepdims=True)
        better = max_sum * inv_n > bestfit_ref[0, 0]
        add_ref[...] = jnp.where(better, chosen, bestpos_ref[...])

    @pl.when(phase == 1)
    def _():
        out_ref[...] = xs_ref[pl.ds(i * _BLK, _BLK), :] + add_ref[...]


def kernel(x, bee_positions, bee_fitness, best_position, best_fitness,
           exploration_radius, exploitation_radius):
    B, S, H = x.shape
    n_rows = B * S
    xr = x.reshape(n_rows, H)
    n_blk = n_rows // _BLK
    body = functools.partial(_bee_kernel, inv_n=1.0 / n_rows)

    out = pl.pallas_call(
        body,
        grid=(2, n_blk),
        in_specs=[
            pl.BlockSpec((_NUM_BEES, H), lambda p, i: (0, 0)),
            pl.BlockSpec((1, H), lambda p, i: (0, 0)),
            pl.BlockSpec((1, 1), lambda p, i: (0, 0)),
            pl.BlockSpec((_BLK, H),
                         lambda p, i: (i * (1 - p) + (n_blk - 1) * p, 0)),
        ],
        out_specs=pl.BlockSpec((_BLK, H), lambda p, i: (i * p, 0)),
        out_shape=jax.ShapeDtypeStruct((n_rows, H), jnp.float32),
        scratch_shapes=[
            pltpu.VMEM((n_rows, H), jnp.float32),
            pltpu.VMEM((1, _NUM_BEES), jnp.float32),
            pltpu.VMEM((1, H), jnp.float32),
        ],
    )(bee_positions, best_position.reshape(1, H),
      best_fitness.reshape(1, 1), xr)

    return out.reshape(B, S, H)
